# baseline (speedup 1.0000x reference)
import functools
import math

import jax
import jax.numpy as jnp
from jax import lax
from jax.experimental import pallas as pl
from jax.experimental.pallas import tpu as pltpu
from jax.experimental.pallas import tpu_sc as plsc

D_MODEL = 64
SCALE = math.sqrt(D_MODEL)

NC = 2
NS = 16
NW = NC * NS

B_TOT = 4096 * 50
B_PER_W = B_TOT // NW
CHUNK = 400
NCHUNK = B_PER_W // CHUNK
NBLK = CHUNK // 16

_mesh = plsc.VectorSubcoreMesh(core_axis_name="c", subcore_axis_name="s")


@functools.partial(
    pl.kernel,
    mesh=_mesh,
    out_type=jax.ShapeDtypeStruct((B_TOT // 2, 128), jnp.float32),
    scratch_types=[
        pltpu.VMEM((CHUNK,), jnp.int32),
        pltpu.VMEM((CHUNK,), jnp.int32),
        pltpu.VMEM((CHUNK, 128), jnp.float32),
        pltpu.VMEM((CHUNK // 2, 128), jnp.float32),
        pltpu.SemaphoreType.DMA,
    ],
    compiler_params=pltpu.CompilerParams(needs_layout_passes=False),
)
def _embed(x_hbm, table_hbm, out_hbm, idx_v, pair_v, gath_v, out_v, sem):
    wid = lax.axis_index("s") * NC + lax.axis_index("c")
    base = wid * B_PER_W
    iota = lax.iota(jnp.int32, 16)

    def chunk_body(j, carry):
        off = pl.multiple_of(base + j * CHUNK, CHUNK)
        pltpu.sync_copy(x_hbm.at[pl.ds(off, CHUNK)], idx_v)

        def halve(b, c):
            s = pl.ds(b * 16, 16)
            pair_v[s] = lax.shift_right_logical(idx_v[s], 1)
            return c

        lax.fori_loop(0, NBLK, halve, 0)
        pltpu.async_copy(table_hbm.at[pair_v], gath_v, sem).wait()

        def reshuffle(b, c):
            iv = idx_v[pl.ds(b * 16, 16)]
            hv = (iv & 1) * 64
            rows = iota + b * 16
            orow = (iota >> 1) + b * 8
            ocol = (iota & 1) * 64
            for col in range(D_MODEL):
                vals = plsc.load_gather(gath_v, [rows, hv + col])
                plsc.store_scatter(out_v, [orow, ocol + col], vals * SCALE)
            return c

        lax.fori_loop(0, NBLK, reshuffle, 0)
        pltpu.sync_copy(
            out_v, out_hbm.at[pl.ds(pl.multiple_of(off // 2, CHUNK // 2), CHUNK // 2)]
        )
        return carry

    lax.fori_loop(0, NCHUNK, chunk_body, 0)


def kernel(x, table):
    out = _embed(x.reshape(-1), table.reshape(500000, 128))
    return out.reshape(x.shape + (D_MODEL,))


# speedup vs baseline: 1.5085x; 1.5085x over previous
import functools
import math

import jax
import jax.numpy as jnp
from jax import lax
from jax.experimental import pallas as pl
from jax.experimental.pallas import tpu as pltpu
from jax.experimental.pallas import tpu_sc as plsc

D_MODEL = 64
SCALE = math.sqrt(D_MODEL)

NC = 2
NS = 16
NW = NC * NS

B_TOT = 4096 * 50
B_PER_W = B_TOT // NW
CHUNK = 400
NCHUNK = B_PER_W // CHUNK

_mesh = plsc.VectorSubcoreMesh(core_axis_name="c", subcore_axis_name="s")


@functools.partial(
    pl.kernel,
    mesh=_mesh,
    out_type=jax.ShapeDtypeStruct((B_TOT // 2, 128), jnp.float32),
    scratch_types=[
        pltpu.VMEM((CHUNK,), jnp.int32),
        pltpu.VMEM((CHUNK, 128), jnp.float32),
        pltpu.VMEM((CHUNK // 2, 128), jnp.float32),
        pltpu.SemaphoreType.DMA,
    ],
    compiler_params=pltpu.CompilerParams(needs_layout_passes=False),
)
def _embed(x_hbm, table_hbm, out_hbm, idx_v, gath_v, out_v, sem):
    wid = lax.axis_index("s") * NC + lax.axis_index("c")
    base = wid * B_PER_W

    def chunk_body(j, carry):
        off = pl.multiple_of(base + j * CHUNK, CHUNK)
        pltpu.sync_copy(x_hbm.at[pl.ds(off, CHUNK)], idx_v)
        pltpu.async_copy(table_hbm.at[idx_v], gath_v, sem).wait()

        def scale_row(r, c):
            half = (r & 1) * D_MODEL
            for q in range(D_MODEL // 16):
                out_v[r >> 1, pl.ds(half + q * 16, 16)] = (
                    gath_v[r, pl.ds(q * 16, 16)] * SCALE
                )
            return c

        lax.fori_loop(0, CHUNK, scale_row, 0)
        pltpu.sync_copy(
            out_v,
            out_hbm.at[pl.ds(pl.multiple_of(off // 2, CHUNK // 2), CHUNK // 2)],
        )
        return carry

    lax.fori_loop(0, NCHUNK, chunk_body, 0)


def kernel(x, table):
    padded = jnp.pad(table, ((0, 0), (0, 128 - D_MODEL)))
    out = _embed(x.reshape(-1), padded)
    return out.reshape(x.shape + (D_MODEL,))


# speedup vs baseline: 1.6141x; 1.0700x over previous
import functools
import math

import jax
import jax.numpy as jnp
from jax import lax
from jax.experimental import pallas as pl
from jax.experimental.pallas import tpu as pltpu
from jax.experimental.pallas import tpu_sc as plsc

D_MODEL = 64
SCALE = math.sqrt(D_MODEL)

NC = 2
NS = 16
NW = NC * NS

B_TOT = 4096 * 50
B_PER_W = B_TOT // NW
CHUNK = 320
NCHUNK = B_PER_W // CHUNK

_mesh = plsc.VectorSubcoreMesh(core_axis_name="c", subcore_axis_name="s")


@functools.partial(
    pl.kernel,
    mesh=_mesh,
    out_type=jax.ShapeDtypeStruct((B_TOT // 2, 128), jnp.float32),
    scratch_types=[
        pltpu.VMEM((CHUNK,), jnp.int32),
        pltpu.VMEM((CHUNK,), jnp.int32),
        pltpu.VMEM((2, CHUNK, 128), jnp.float32),
        pltpu.VMEM((2, CHUNK // 2, 128), jnp.float32),
        pltpu.SemaphoreType.DMA,
        pltpu.SemaphoreType.DMA,
        pltpu.SemaphoreType.DMA,
        pltpu.SemaphoreType.DMA,
    ],
    compiler_params=pltpu.CompilerParams(needs_layout_passes=False),
)
def _embed(x_hbm, table_hbm, out_hbm, idx0_v, idx1_v, gath_v, out_v,
           isem0, isem1, gsem, osem):
    wid = lax.axis_index("s") * NC + lax.axis_index("c")
    base = wid * B_PER_W
    idxs = (idx0_v, idx1_v)
    isems = (isem0, isem1)

    def start_fetch(j, b):
        off = pl.multiple_of(base + j * CHUNK, CHUNK)
        pltpu.async_copy(x_hbm.at[pl.ds(off, CHUNK)], idxs[b], isems[b]
                         ).wait()
        pltpu.async_copy(table_hbm.at[idxs[b]], gath_v.at[b], gsem)

    def drain_gather(b):
        pltpu.make_async_copy(table_hbm.at[idxs[b]], gath_v.at[b], gsem
                              ).wait()

    def compact(j, b):
        def pair(r2, c):
            for sub in range(2):
                for q in range(D_MODEL // 16):
                    out_v[b, r2, pl.ds(sub * D_MODEL + q * 16, 16)] = (
                        gath_v[b, 2 * r2 + sub, pl.ds(q * 16, 16)] * SCALE
                    )
            return c

        lax.fori_loop(0, CHUNK // 2, pair, 0)

    def store_out(j, b):
        off2 = pl.multiple_of((base + j * CHUNK) // 2, CHUNK // 2)
        pltpu.async_copy(out_v.at[b], out_hbm.at[pl.ds(off2, CHUNK // 2)],
                         osem)

    def drain_out(j, b):
        off2 = pl.multiple_of((base + j * CHUNK) // 2, CHUNK // 2)
        pltpu.make_async_copy(out_v.at[b], out_hbm.at[pl.ds(off2, CHUNK // 2)],
                              osem).wait()

    start_fetch(0, 0)

    def loop(j2, carry):
        j = j2 * 2
        start_fetch(j + 1, 1)
        drain_gather(0)
        compact(j, 0)
        lax.cond(j2 > 0, lambda: drain_out(j - 1, 1), lambda: None)
        store_out(j, 0)
        lax.cond(j2 + 1 < NCHUNK // 2,
                 lambda: start_fetch(j + 2, 0), lambda: None)
        drain_gather(1)
        compact(j + 1, 1)
        drain_out(j, 0)
        store_out(j + 1, 1)
        return carry

    lax.fori_loop(0, NCHUNK // 2, loop, 0)
    drain_out(NCHUNK - 1, 1)


def kernel(x, table):
    padded = jnp.pad(table, ((0, 0), (0, 128 - D_MODEL)))
    out = _embed(x.reshape(-1), padded)
    return out.reshape(x.shape + (D_MODEL,))


# speedup vs baseline: 1.7895x; 1.1087x over previous
import functools
import math

import jax
import jax.numpy as jnp
from jax import lax
from jax.experimental import pallas as pl
from jax.experimental.pallas import tpu as pltpu
from jax.experimental.pallas import tpu_sc as plsc

D_MODEL = 64
SCALE = math.sqrt(D_MODEL)

NC = 2
NS = 16
NW = NC * NS

B_TOT = 4096 * 50
B_PER_W = B_TOT // NW
CHUNK = 320
NCHUNK = B_PER_W // CHUNK

_mesh = plsc.VectorSubcoreMesh(core_axis_name="c", subcore_axis_name="s")


@functools.partial(
    pl.kernel,
    mesh=_mesh,
    out_type=jax.ShapeDtypeStruct((B_TOT // 2, 128), jnp.float32),
    scratch_types=[
        pltpu.VMEM((CHUNK,), jnp.int32),
        pltpu.VMEM((CHUNK,), jnp.int32),
        pltpu.VMEM((2, CHUNK, 128), jnp.float32),
        pltpu.VMEM((2, CHUNK // 2, 128), jnp.float32),
        pltpu.SemaphoreType.DMA,
        pltpu.SemaphoreType.DMA,
        pltpu.SemaphoreType.DMA,
        pltpu.SemaphoreType.DMA,
    ],
    compiler_params=pltpu.CompilerParams(needs_layout_passes=False),
)
def _embed(x_hbm, table_hbm, out_hbm, idx0_v, idx1_v, gath_v, out_v,
           isem0, isem1, gsem, osem):
    wid = lax.axis_index("s") * NC + lax.axis_index("c")
    base = wid * B_PER_W
    idxs = (idx0_v, idx1_v)
    isems = (isem0, isem1)

    def start_fetch(j, b):
        off = pl.multiple_of(base + j * CHUNK, CHUNK)
        pltpu.async_copy(x_hbm.at[pl.ds(off, CHUNK)], idxs[b], isems[b]
                         ).wait()
        pltpu.async_copy(table_hbm.at[idxs[b]], gath_v.at[b], gsem)

    def drain_gather(b):
        pltpu.make_async_copy(table_hbm.at[idxs[b]], gath_v.at[b], gsem
                              ).wait()

    def compact(j, b):
        @plsc.parallel_loop(0, CHUNK // 2, unroll=4)
        def pair(r2):
            for sub in range(2):
                for q in range(D_MODEL // 16):
                    out_v[b, r2, pl.ds(sub * D_MODEL + q * 16, 16)] = (
                        gath_v[b, 2 * r2 + sub, pl.ds(q * 16, 16)] * SCALE
                    )

    def store_out(j, b):
        off2 = pl.multiple_of((base + j * CHUNK) // 2, CHUNK // 2)
        pltpu.async_copy(out_v.at[b], out_hbm.at[pl.ds(off2, CHUNK // 2)],
                         osem)

    def drain_out(j, b):
        off2 = pl.multiple_of((base + j * CHUNK) // 2, CHUNK // 2)
        pltpu.make_async_copy(out_v.at[b], out_hbm.at[pl.ds(off2, CHUNK // 2)],
                              osem).wait()

    start_fetch(0, 0)

    def loop(j2, carry):
        j = j2 * 2
        start_fetch(j + 1, 1)
        drain_gather(0)
        compact(j, 0)
        lax.cond(j2 > 0, lambda: drain_out(j - 1, 1), lambda: None)
        store_out(j, 0)
        lax.cond(j2 + 1 < NCHUNK // 2,
                 lambda: start_fetch(j + 2, 0), lambda: None)
        drain_gather(1)
        compact(j + 1, 1)
        drain_out(j, 0)
        store_out(j + 1, 1)
        return carry

    lax.fori_loop(0, NCHUNK // 2, loop, 0)
    drain_out(NCHUNK - 1, 1)


def kernel(x, table):
    padded = jnp.pad(table, ((0, 0), (0, 128 - D_MODEL)))
    out = _embed(x.reshape(-1), padded)
    return out.reshape(x.shape + (D_MODEL,))
